# baseline (device time: 23368 ns/iter reference)
import jax
import jax.numpy as jnp
from jax import lax
from jax.experimental import pallas as pl
from jax.experimental.pallas import tpu as pltpu


def kernel(Q, K, V):
    B, S, H, D = Q.shape
    BH = B * H
    scale = D ** -0.5

    x_idx = lax.axis_index("x")

    def prep(A, s):
        Ah = lax.dynamic_slice_in_dim(A, x_idx, 1, axis=0)[0]
        return jnp.transpose((Ah * s).astype(jnp.bfloat16), (1, 2, 0))

    Qb = prep(Q, scale)
    Kb = prep(K, 1.0)
    Vb = prep(V, 1.0)

    def body(q_ref, k_ref, v_ref, o_ref, kbuf, vbuf, oacc, lacc,
             kv_send, kv_recv, o_send, o_recv, xbar_sem):
        my_x = lax.axis_index("x")
        my_y = lax.axis_index("y")
        ypeer = (my_x, 1 - my_y)
        xpeer = (1 - my_x, my_y)
        base = my_x * H

        with jax.named_scope("barrier"):
            barrier_sem = pltpu.get_barrier_semaphore()
            pl.semaphore_signal(
                barrier_sem, inc=1, device_id=ypeer,
                device_id_type=pl.DeviceIdType.MESH,
            )
            pl.semaphore_wait(barrier_sem, 1)

        rkvs = []
        for j in range(H):
            rk = pltpu.make_async_remote_copy(
                src_ref=k_ref.at[j], dst_ref=kbuf.at[j],
                send_sem=kv_send.at[2 * j], recv_sem=kv_recv.at[2 * j],
                device_id=ypeer, device_id_type=pl.DeviceIdType.MESH,
            )
            rk.start()
            rv = pltpu.make_async_remote_copy(
                src_ref=v_ref.at[j], dst_ref=vbuf.at[j],
                send_sem=kv_send.at[2 * j + 1],
                recv_sem=kv_recv.at[2 * j + 1],
                device_id=ypeer, device_id_type=pl.DeviceIdType.MESH,
            )
            rv.start()
            rkvs.append((rk, rv))

        pl.semaphore_signal(
            xbar_sem, inc=1, device_id=xpeer,
            device_id_type=pl.DeviceIdType.MESH,
        )

        ones = jnp.ones((S,), jnp.bfloat16)

        def phase(kref, vref, j):
            pT = jnp.exp(
                lax.dot_general(
                    kref[j], q_ref[j], (((0,), (0,)), ((), ())),
                    preferred_element_type=jnp.float32,
                )
            ).astype(jnp.bfloat16)
            l = lax.dot_general(
                pT, ones, (((0,), (0,)), ((), ())),
                preferred_element_type=jnp.float32,
            )
            oT = lax.dot_general(
                vref[j], pT, (((1,), (0,)), ((), ())),
                preferred_element_type=jnp.float32,
            )
            return l, oT

        with jax.named_scope("local_phase"):
            for j in range(H):
                l0, o0 = phase(k_ref, v_ref, j)
                lacc[j] = l0
                oacc[j] = o0

        ros = []
        with jax.named_scope("remote_phase"):
            for j in range(H):
                rkvs[j][0].wait_recv()
                rkvs[j][1].wait_recv()
                if j == 0:
                    pl.semaphore_wait(xbar_sem, 1)
                l1, o1 = phase(kbuf, vbuf, j)
                r = (oacc[j] + o1) / (lacc[j] + l1)[None, :]
                o_ref[base + j] = r.astype(jnp.bfloat16)
                ro = pltpu.make_async_remote_copy(
                    src_ref=o_ref.at[base + j],
                    dst_ref=o_ref.at[base + j],
                    send_sem=o_send.at[j], recv_sem=o_recv.at[j],
                    device_id=xpeer, device_id_type=pl.DeviceIdType.MESH,
                )
                ro.start()
                ros.append(ro)

        with jax.named_scope("out_flush"):
            for j in range(H):
                ros[j].wait_recv()
            for j in range(H):
                ros[j].wait_send()
                rkvs[j][0].wait_send()
                rkvs[j][1].wait_send()

    out = pl.pallas_call(
        body,
        out_shape=jax.ShapeDtypeStruct((BH, D, S), jnp.bfloat16),
        in_specs=[pl.BlockSpec(memory_space=pltpu.VMEM)] * 3,
        out_specs=pl.BlockSpec(memory_space=pltpu.VMEM),
        scratch_shapes=[
            pltpu.VMEM((H, D, S), jnp.bfloat16),
            pltpu.VMEM((H, D, S), jnp.bfloat16),
            pltpu.VMEM((H, D, S), jnp.float32),
            pltpu.VMEM((H, S), jnp.float32),
            pltpu.SemaphoreType.DMA((2 * H,)),
            pltpu.SemaphoreType.DMA((2 * H,)),
            pltpu.SemaphoreType.DMA((H,)),
            pltpu.SemaphoreType.DMA((H,)),
            pltpu.SemaphoreType.REGULAR,
        ],
        compiler_params=pltpu.CompilerParams(collective_id=0),
    )(Qb, Kb, Vb)

    return out.reshape(B, H, D, S).transpose(0, 3, 1, 2)
